# baseline (speedup 1.0000x reference)
import jax
import jax.numpy as jnp
from jax.experimental import pallas as pl
from jax.experimental.pallas import tpu as pltpu

_E = 16
_H = 1024
_FF = 1024
_ALPHA = 1.702
_LIMIT = 7.0
_FFB = 1024
_NFF = _FF // _FFB
_NTOK = 128


def _moe_kernel(x_ref, rw_ref, rb_ref, gw_ref, gb_ref, uw_ref, ub_ref,
                dw_ref, db_ref, out_ref, scores_ref, scores_scr):
    e = pl.program_id(0)
    j = pl.program_id(1)

    @pl.when((e == 0) & (j == 0))
    def _router():
        x = x_ref[...]
        logits = jax.lax.dot_general(
            x, rw_ref[...], (((1,), (1,)), ((), ())),
            preferred_element_type=jnp.float32) + rb_ref[0][None, :]
        cols = jax.lax.broadcasted_iota(jnp.int32, logits.shape, 1)
        i1 = jnp.argmax(logits, axis=1)
        m1 = jnp.max(logits, axis=1)
        masked = jnp.where(cols == i1[:, None], -jnp.inf, logits)
        i2 = jnp.argmax(masked, axis=1)
        m2 = jnp.max(masked, axis=1)
        t = jnp.exp(m2 - m1)
        p1 = 1.0 / (1.0 + t)
        p2 = t / (1.0 + t)
        scores = (jnp.where(cols == i1[:, None], p1[:, None], 0.0)
                  + jnp.where(cols == i2[:, None], p2[:, None], 0.0))
        scores_scr[...] = scores
        scores_ref[...] = scores
        out_ref[...] = jnp.zeros_like(out_ref)

    x = x_ref[...]
    cols = jax.lax.broadcasted_iota(jnp.int32, (_NTOK, _E), 1)
    s = jnp.sum(jnp.where(cols == e, scores_scr[...], 0.0), axis=1,
                keepdims=True)

    xb = x.astype(jnp.bfloat16)
    gate = jax.lax.dot_general(
        xb, gw_ref[0].astype(jnp.bfloat16), (((1,), (1,)), ((), ())),
        preferred_element_type=jnp.float32) + gb_ref[0]
    up = jax.lax.dot_general(
        xb, uw_ref[0].astype(jnp.bfloat16), (((1,), (1,)), ((), ())),
        preferred_element_type=jnp.float32) + ub_ref[0]
    gate = jnp.minimum(gate, _LIMIT)
    up = jnp.clip(up, -_LIMIT, _LIMIT)
    glu = gate * jax.nn.sigmoid(gate * _ALPHA)
    act = ((up + 1.0) * glu).astype(jnp.bfloat16)
    y = jax.lax.dot_general(
        act, dw_ref[0].astype(jnp.bfloat16), (((1,), (1,)), ((), ())),
        preferred_element_type=jnp.float32)
    y = y * s
    y = jnp.where(j == 0, y + s * db_ref[0], y)
    out_ref[...] += y


def kernel(hidden_states, router_w, router_b, gate_w, gate_b, up_w, up_b,
           down_w, down_b):
    Bn, Tn, Hn = hidden_states.shape
    x = hidden_states.reshape(-1, Hn)
    rb2 = router_b.reshape(1, _E)
    gb3 = gate_b.reshape(_E, 1, _FF)
    ub3 = up_b.reshape(_E, 1, _FF)
    db3 = down_b.reshape(_E, 1, _H)

    out, scores = pl.pallas_call(
        _moe_kernel,
        grid=(_E, _NFF),
        in_specs=[
            pl.BlockSpec((_NTOK, _H), lambda e, j: (0, 0)),
            pl.BlockSpec((_E, _H), lambda e, j: (0, 0)),
            pl.BlockSpec((1, _E), lambda e, j: (0, 0)),
            pl.BlockSpec((1, _FFB, _H), lambda e, j: (e, j, 0)),
            pl.BlockSpec((1, 1, _FFB), lambda e, j: (e, 0, j)),
            pl.BlockSpec((1, _FFB, _H), lambda e, j: (e, j, 0)),
            pl.BlockSpec((1, 1, _FFB), lambda e, j: (e, 0, j)),
            pl.BlockSpec((1, _H, _FFB), lambda e, j: (e, 0, j)),
            pl.BlockSpec((1, 1, _H), lambda e, j: (e, 0, 0)),
        ],
        out_specs=[
            pl.BlockSpec((_NTOK, _H), lambda e, j: (0, 0)),
            pl.BlockSpec((_NTOK, _E), lambda e, j: (0, 0)),
        ],
        out_shape=[
            jax.ShapeDtypeStruct((_NTOK, _H), jnp.float32),
            jax.ShapeDtypeStruct((_NTOK, _E), jnp.float32),
        ],
        scratch_shapes=[pltpu.VMEM((_NTOK, _E), jnp.float32)],
        compiler_params=pltpu.CompilerParams(
            dimension_semantics=("arbitrary", "arbitrary")),
    )(x, router_w, rb2, gate_w, gb3, up_w, ub3, down_w, db3)

    return out.reshape(Bn, Tn, Hn), scores


# speedup vs baseline: 1.0547x; 1.0547x over previous
import jax
import jax.numpy as jnp
from jax.experimental import pallas as pl
from jax.experimental.pallas import tpu as pltpu

_E = 16
_H = 1024
_FF = 1024
_ALPHA = 1.702
_LIMIT = 7.0
_NTOK = 128


def _moe_kernel(x_ref, rw_ref, rb_ref, bias_ref, gw_ref, uw_ref, dw_ref,
                out_ref, scores_ref, scores_scr):
    e = pl.program_id(0)

    @pl.when(e == 0)
    def _router():
        x = x_ref[...]
        logits = jax.lax.dot_general(
            x, rw_ref[...], (((1,), (1,)), ((), ())),
            preferred_element_type=jnp.float32) + rb_ref[0][None, :]
        cols = jax.lax.broadcasted_iota(jnp.int32, logits.shape, 1)
        i1 = jnp.argmax(logits, axis=1)
        m1 = jnp.max(logits, axis=1)
        masked = jnp.where(cols == i1[:, None], -jnp.inf, logits)
        i2 = jnp.argmax(masked, axis=1)
        m2 = jnp.max(masked, axis=1)
        t = jnp.exp(m2 - m1)
        p1 = 1.0 / (1.0 + t)
        p2 = t / (1.0 + t)
        scores = (jnp.where(cols == i1[:, None], p1[:, None], 0.0)
                  + jnp.where(cols == i2[:, None], p2[:, None], 0.0))
        scores_scr[...] = scores
        scores_ref[...] = scores

    x = x_ref[...]
    cols = jax.lax.broadcasted_iota(jnp.int32, (_NTOK, _E), 1)
    s = jnp.sum(jnp.where(cols == e, scores_scr[...], 0.0), axis=1,
                keepdims=True)

    gb = bias_ref[pl.ds(e, 1), 0:_FF]
    ub = bias_ref[pl.ds(e, 1), _FF:2 * _FF]
    db = bias_ref[pl.ds(e, 1), 2 * _FF:2 * _FF + _H]

    gate = jax.lax.dot_general(
        x, gw_ref[0], (((1,), (1,)), ((), ())),
        preferred_element_type=jnp.float32) + gb
    up = jax.lax.dot_general(
        x, uw_ref[0], (((1,), (1,)), ((), ())),
        preferred_element_type=jnp.float32) + ub
    gate = jnp.minimum(gate, _LIMIT)
    up = jnp.clip(up, -_LIMIT, _LIMIT)
    glu = gate * jax.nn.sigmoid(gate * _ALPHA)
    act = (up + 1.0) * glu
    y = jax.lax.dot_general(
        act, dw_ref[0], (((1,), (1,)), ((), ())),
        preferred_element_type=jnp.float32)
    y = (y + db) * s

    @pl.when(e == 0)
    def _init():
        out_ref[...] = y

    @pl.when(e != 0)
    def _acc():
        out_ref[...] += y


def kernel(hidden_states, router_w, router_b, gate_w, gate_b, up_w, up_b,
           down_w, down_b):
    Bn, Tn, Hn = hidden_states.shape
    x = hidden_states.reshape(-1, Hn)
    rb2 = router_b.reshape(1, _E)
    biases = jnp.concatenate([gate_b, up_b, down_b], axis=1)

    out, scores = pl.pallas_call(
        _moe_kernel,
        grid=(_E,),
        in_specs=[
            pl.BlockSpec((_NTOK, _H), lambda e: (0, 0)),
            pl.BlockSpec((_E, _H), lambda e: (0, 0)),
            pl.BlockSpec((1, _E), lambda e: (0, 0)),
            pl.BlockSpec((_E, 2 * _FF + _H), lambda e: (0, 0)),
            pl.BlockSpec((1, _FF, _H), lambda e: (e, 0, 0)),
            pl.BlockSpec((1, _FF, _H), lambda e: (e, 0, 0)),
            pl.BlockSpec((1, _H, _FF), lambda e: (e, 0, 0)),
        ],
        out_specs=[
            pl.BlockSpec((_NTOK, _H), lambda e: (0, 0)),
            pl.BlockSpec((_NTOK, _E), lambda e: (0, 0)),
        ],
        out_shape=[
            jax.ShapeDtypeStruct((_NTOK, _H), jnp.float32),
            jax.ShapeDtypeStruct((_NTOK, _E), jnp.float32),
        ],
        scratch_shapes=[pltpu.VMEM((_NTOK, _E), jnp.float32)],
        compiler_params=pltpu.CompilerParams(
            dimension_semantics=("arbitrary",)),
    )(x, router_w, rb2, biases, gate_w, up_w, down_w)

    return out.reshape(Bn, Tn, Hn), scores
